# baseline (speedup 1.0000x reference)
import jax
import jax.numpy as jnp
from jax import lax
from jax.experimental import pallas as pl
from jax.experimental.pallas import tpu as pltpu
from jax.experimental.pallas import tpu_sc as plsc

D = 32
L = 200
B = 4096
N = B * L
V = 1000000
NC, NS = 2, 16
NW = NC * NS
BPW = B // NW
PER_W = N // NW
RPC = 8
CHUNK = RPC * L
NCH = BPW // RPC
SUB = 80
NSUB = CHUNK // SUB
LANES = 16
_COLS = [k * LANES for k in range(L // LANES)] + [L - LANES]


def _flatten_body(x_hbm, out_hbm, xb, xf, sem):
    wid = lax.axis_index("s") * NC + lax.axis_index("c")
    b0 = pl.multiple_of(wid * BPW, BPW)
    pltpu.sync_copy(x_hbm.at[pl.ds(b0, BPW)], xb)

    def row(r, _):
        for col in _COLS:
            xf[pl.ds(r * L + col, LANES)] = xb[r, pl.ds(col, LANES)]
        return 0

    lax.fori_loop(0, BPW, row, 0)
    pltpu.sync_copy(xf, out_hbm.at[pl.ds(wid * PER_W, PER_W)])


def _gather_body(x_hbm, table_hbm, pos_hbm, out_hbm, idx_v, gbuf, dest_v,
                 pos_v, sem):
    wid = lax.axis_index("s") * NC + lax.axis_index("c")
    base_w = pl.multiple_of(wid * PER_W, PER_W)
    pltpu.sync_copy(pos_hbm, pos_v)

    def chunk_body(ch, _):
        base = pl.multiple_of(base_w + ch * CHUNK, CHUNK)
        pltpu.sync_copy(x_hbm.at[pl.ds(base, CHUNK)], idx_v)
        copies = [
            pltpu.async_copy(
                table_hbm.at[idx_v.at[pl.ds(j * SUB, SUB)]],
                gbuf.at[pl.ds(j * SUB, SUB)],
                sem,
            )
            for j in range(NSUB)
        ]
        for cp in copies:
            cp.wait()

        def add_l(l, _):
            p0 = pos_v[l, pl.ds(0, LANES)]
            p1 = pos_v[l, pl.ds(LANES, LANES)]
            for t in range(RPC):
                r2 = t * L + l
                dest_v[pl.ds(r2 * D, LANES)] = gbuf[r2, pl.ds(0, LANES)] + p0
                dest_v[pl.ds(r2 * D + LANES, LANES)] = (
                    gbuf[r2, pl.ds(LANES, LANES)] + p1)
            return 0

        lax.fori_loop(0, L, add_l, 0)
        pltpu.sync_copy(dest_v, out_hbm.at[pl.ds(base * D, CHUNK * D)])
        return 0

    lax.fori_loop(0, NCH, chunk_body, 0)


_mesh = plsc.VectorSubcoreMesh(core_axis_name="c", subcore_axis_name="s")

_flatten = pl.kernel(
    _flatten_body,
    out_type=jax.ShapeDtypeStruct((N,), jnp.int32),
    mesh=_mesh,
    scratch_types=[
        pltpu.VMEM((BPW, L), jnp.int32),
        pltpu.VMEM((PER_W,), jnp.int32),
        pltpu.SemaphoreType.DMA,
    ],
)

_gather = pl.kernel(
    _gather_body,
    out_type=jax.ShapeDtypeStruct((N * D,), jnp.float32),
    mesh=_mesh,
    scratch_types=[
        pltpu.VMEM((CHUNK,), jnp.int32),
        pltpu.VMEM((CHUNK, D), jnp.float32),
        pltpu.VMEM((CHUNK * D,), jnp.float32),
        pltpu.VMEM((L, D), jnp.float32),
        pltpu.SemaphoreType.DMA,
    ],
    compiler_params=pltpu.CompilerParams(use_tc_tiling_on_sc=False),
)


@jax.jit
def _run(x, table, pos):
    xf = _flatten(x)
    out = _gather(xf, table, pos)
    return out.reshape(B, L, D)


def kernel(x, table, pos):
    return _run(x, table, pos)
